# baseline (device time: 15249 ns/iter reference)
import jax
import jax.numpy as jnp
from jax import lax
from jax.experimental import pallas as pl
from jax.experimental.pallas import tpu as pltpu

NC = 16


def kernel(x, dest):
    t, d = x.shape
    c = t // NC
    dest2 = dest.reshape(1, t).astype(jnp.int32)

    def body(x_ref, dest_ref, out_ref, xs, xloc, px, send_sems, recv_sems):
        p = lax.axis_index("y")
        mx = lax.axis_index("x")
        mz = lax.axis_index("z")
        peer = (mx, 1 - p, mz)

        bar = pltpu.get_barrier_semaphore()
        pl.semaphore_signal(
            bar, inc=1, device_id=peer, device_id_type=pl.DeviceIdType.MESH
        )

        xs[...] = x_ref[...].astype(jnp.bfloat16)

        md = dest_ref[...] == p
        mdi = md.astype(jnp.int32)
        j_vec = lax.broadcasted_iota(jnp.int32, (1, t), 1)
        v = mdi
        s = 1
        while s < t:
            v = v + jnp.where(j_vec >= s, pltpu.roll(v, s, 1), 0)
            s *= 2
        n_self = jnp.sum(mdi)
        m = t - n_self
        c_keep = v - mdi
        c_send = j_vec - v + mdi
        rank = jnp.where(md, m + c_keep, c_send)

        pl.semaphore_wait(bar, 1)

        rdmas = []
        for k in range(NC):
            rdmas.append(
                pltpu.make_async_remote_copy(
                    src_ref=xloc.at[pl.ds(k * c, c)],
                    dst_ref=px.at[pl.ds(k * c, c)],
                    send_sem=send_sems.at[k],
                    recv_sem=recv_sems.at[k],
                    device_id=peer,
                    device_id_type=pl.DeviceIdType.MESH,
                )
            )
        h = t // 2
        kh = NC // 2
        for half in range(2):
            i_io = lax.broadcasted_iota(jnp.int32, (h, t), 0) + half * h
            p_h = (i_io == rank).astype(jnp.bfloat16)
            xloc[pl.ds(half * h, h), :] = jnp.dot(
                p_h, xs[...], preferred_element_type=jnp.float32
            ).astype(jnp.bfloat16)
            for k in range(half * kh, (half + 1) * kh):
                @pl.when(k * c < m)
                def _(k=k):
                    rdmas[k].start()

        shift = (1 - p) * n_self
        row_i = lax.broadcasted_iota(jnp.int32, (t, 1), 0)
        zero = jnp.array(0, jnp.bfloat16)
        keep_rolled = pltpu.roll(
            jnp.where(row_i < m, zero, xloc[...]), shift, 0
        )

        for k in range(NC):
            @pl.when(k * c < m)
            def _(k=k):
                rdmas[k].wait_recv()

        recv_rolled = pltpu.roll(
            jnp.where(row_i < m, px[...], zero), shift, 0
        )
        out_ref[...] = keep_rolled + recv_rolled

        for k in range(NC):
            @pl.when(k * c < m)
            def _(k=k):
                rdmas[k].wait_send()

    return pl.pallas_call(
        body,
        out_shape=jax.ShapeDtypeStruct((t, d), jnp.bfloat16),
        in_specs=[
            pl.BlockSpec(memory_space=pltpu.VMEM),
            pl.BlockSpec(memory_space=pltpu.VMEM),
        ],
        out_specs=pl.BlockSpec(memory_space=pltpu.VMEM),
        scratch_shapes=[
            pltpu.VMEM((t, d), jnp.bfloat16),
            pltpu.VMEM((t, d), jnp.bfloat16),
            pltpu.VMEM((t, d), jnp.bfloat16),
            pltpu.SemaphoreType.DMA((NC,)),
            pltpu.SemaphoreType.DMA((NC,)),
        ],
        compiler_params=pltpu.CompilerParams(collective_id=0),
    )(x, dest2)


# device time: 14627 ns/iter; 1.0425x vs baseline; 1.0425x over previous
import jax
import jax.numpy as jnp
from jax import lax
from jax.experimental import pallas as pl
from jax.experimental.pallas import tpu as pltpu

NC = 16


def kernel(x, dest):
    t, d = x.shape
    c = t // NC

    def body(x_ref, dest_ref, out_ref, xs, xloc, px, send_sems, recv_sems):
        p = lax.axis_index("y")
        mx = lax.axis_index("x")
        mz = lax.axis_index("z")
        peer = (mx, 1 - p, mz)

        bar = pltpu.get_barrier_semaphore()
        pl.semaphore_signal(
            bar, inc=1, device_id=peer, device_id_type=pl.DeviceIdType.MESH
        )

        xs[...] = x_ref[...].astype(jnp.bfloat16)

        d2 = dest_ref[...].reshape(1, t)
        md = d2 == p
        mdi = md.astype(jnp.int32)
        mdf = md.astype(jnp.float32)
        tri = (
            lax.broadcasted_iota(jnp.int32, (t, t), 0)
            <= lax.broadcasted_iota(jnp.int32, (t, t), 1)
        ).astype(jnp.float32)
        v = jnp.dot(mdf, tri, preferred_element_type=jnp.float32).astype(
            jnp.int32
        )
        j_vec = lax.broadcasted_iota(jnp.int32, (1, t), 1)
        n_self = jnp.sum(mdi)
        m = t - n_self
        c_keep = v - mdi
        c_send = j_vec - v + mdi
        rank = jnp.where(md, m + c_keep, c_send)

        pl.semaphore_wait(bar, 1)

        rdmas = []
        for k in range(NC):
            rk = pltpu.make_async_remote_copy(
                src_ref=xloc.at[pl.ds(k * c, c)],
                dst_ref=px.at[pl.ds(k * c, c)],
                send_sem=send_sems.at[k],
                recv_sem=recv_sems.at[k],
                device_id=peer,
                device_id_type=pl.DeviceIdType.MESH,
            )
            rdmas.append(rk)
            i_io = lax.broadcasted_iota(jnp.int32, (c, t), 0) + k * c
            p_k = (i_io == rank).astype(jnp.bfloat16)
            xloc[pl.ds(k * c, c), :] = jnp.dot(
                p_k, xs[...], preferred_element_type=jnp.float32
            ).astype(jnp.bfloat16)

            @pl.when(k * c < m)
            def _(rk=rk):
                rk.start()

        shift = (1 - p) * n_self
        row_i = lax.broadcasted_iota(jnp.int32, (t, 1), 0)
        zero = jnp.array(0, jnp.bfloat16)
        keep_rolled = pltpu.roll(
            jnp.where(row_i < m, zero, xloc[...]), shift, 0
        )

        for k in range(NC):
            @pl.when(k * c < m)
            def _(k=k):
                rdmas[k].wait_recv()

        recv_rolled = pltpu.roll(
            jnp.where(row_i < m, px[...], zero), shift, 0
        )
        out_ref[...] = keep_rolled + recv_rolled

        for k in range(NC):
            @pl.when(k * c < m)
            def _(k=k):
                rdmas[k].wait_send()

    return pl.pallas_call(
        body,
        out_shape=jax.ShapeDtypeStruct((t, d), jnp.bfloat16),
        in_specs=[
            pl.BlockSpec(memory_space=pltpu.VMEM),
            pl.BlockSpec(memory_space=pltpu.VMEM),
        ],
        out_specs=pl.BlockSpec(memory_space=pltpu.VMEM),
        scratch_shapes=[
            pltpu.VMEM((t, d), jnp.bfloat16),
            pltpu.VMEM((t, d), jnp.bfloat16),
            pltpu.VMEM((t, d), jnp.bfloat16),
            pltpu.SemaphoreType.DMA((NC,)),
            pltpu.SemaphoreType.DMA((NC,)),
        ],
        compiler_params=pltpu.CompilerParams(collective_id=0),
    )(x, dest)
